# baseline (speedup 1.0000x reference)
import functools

import jax
import jax.numpy as jnp
from jax import lax
from jax.experimental import pallas as pl
from jax.experimental.pallas import tpu as pltpu
from jax.experimental.pallas import tpu_sc as plsc

D = 64
NFIELD = 3
B = 16384
CHUNK = 128

_info = plsc.get_sparse_core_info()
_NC, _NS = _info.num_cores, _info.num_subcores
NW = _NC * _NS
BPW = B // NW
NCHUNK = BPW // CHUNK
L = 16
NGROUP = BPW // L
AHEAD = 4


def _body(uid, iid, cid, wu, wi, wc, ou, oi, oc, idx_v, rows_v, sem):
    wid = lax.axis_index("s") * _NC + lax.axis_index("c")
    base = wid * BPW
    for f, (idx_hbm, table, out) in enumerate(
        ((uid, wu, ou), (iid, wi, oi), (cid, wc, oc))
    ):
        pltpu.sync_copy(idx_hbm.at[pl.ds(wid * NCHUNK, NCHUNK)], idx_v)

        def gather16(g, _, table=table):
            @pl.when(g >= AHEAD)
            def _drain():
                pltpu.make_async_copy(
                    table.at[pl.ds(0, L)], rows_v.at[pl.ds(0, L)], sem
                ).wait()

            v = idx_v[g // (CHUNK // L), pl.ds((g % (CHUNK // L)) * L, L)]
            for k in range(L):
                pltpu.async_copy(
                    table.at[pl.ds(v[k], 1)],
                    rows_v.at[pl.ds(g * L + k, 1)],
                    sem,
                )
            return _

        lax.fori_loop(0, NGROUP, gather16, 0)
        pltpu.make_async_copy(
            table.at[pl.ds(0, AHEAD * L)], rows_v.at[pl.ds(0, AHEAD * L)], sem
        ).wait()
        pltpu.sync_copy(rows_v, out.at[pl.ds(base, BPW)])


@jax.jit
def kernel(user_id, item_id, cat_id, W_user, W_item, W_cat):
    mesh = plsc.VectorSubcoreMesh(core_axis_name="c", subcore_axis_name="s")
    run = functools.partial(
        pl.kernel,
        out_type=[jax.ShapeDtypeStruct((B, D), jnp.float32)] * NFIELD,
        scratch_types=[
            pltpu.VMEM((NCHUNK, CHUNK), jnp.int32),
            pltpu.VMEM((BPW, D), jnp.float32),
            pltpu.SemaphoreType.DMA,
        ],
        mesh=mesh,
    )(_body)
    u = user_id.astype(jnp.int32).reshape(NW * NCHUNK, CHUNK)
    i = item_id.astype(jnp.int32).reshape(NW * NCHUNK, CHUNK)
    c = cat_id.astype(jnp.int32).reshape(NW * NCHUNK, CHUNK)
    ou, oi, oc = run(u, i, c, W_user, W_item, W_cat)
    return jnp.concatenate([ou, oi, oc], axis=-1)
